# Initial kernel scaffold; baseline (speedup 1.0000x reference)
import jax
import jax.numpy as jnp
from jax.experimental import pallas as pl


def kernel(ent_emb, rule_emb, query_relation, head, tail, depth):
    raise NotImplementedError("write your pallas kernel here")


# speedup vs baseline: 1391.9676x; 1391.9676x over previous
import functools

import jax
import jax.numpy as jnp
from jax import lax
from jax.experimental import pallas as pl
from jax.experimental.pallas import tpu as pltpu


def _score_body(ent_ref, rule_ref, qr_ref, depth_ref, head_ref, tail_ref, out_ref):
    E = ent_ref[...]
    R = rule_ref[...]
    N = E.shape[0]
    NR = R.shape[0]
    h = head_ref[...]
    t = tail_ref[...]
    B = h.shape[0]

    zi = lax.broadcasted_iota(jnp.int32, (B, N), 1)
    ohh = (zi == h).astype(jnp.float32)
    oht = (zi == t).astype(jnp.float32)
    H = jnp.dot(ohh, E, preferred_element_type=jnp.float32)
    T = jnp.dot(oht, E, preferred_element_type=jnp.float32)

    dn = (((1,), (1,)), ((), ()))
    GH = lax.dot_general(H, E, dn, preferred_element_type=jnp.float32)
    GT = lax.dot_general(T, E, dn, preferred_element_type=jnp.float32)
    nH = jnp.sum(H * H, axis=1, keepdims=True)
    nT = jnp.sum(T * T, axis=1, keepdims=True)
    ones = jnp.ones((1, E.shape[1]), jnp.float32)
    nE = lax.dot_general(ones, E * E, dn, preferred_element_type=jnp.float32)
    PE = lax.dot_general(R, E, dn, preferred_element_type=jnp.float32)
    PH = lax.dot_general(H, R, dn, preferred_element_type=jnp.float32)

    qr = qr_ref[0]
    rq = rule_ref[pl.ds(qr, 1), :]
    nq = jnp.sum(rq * rq)
    pEq = lax.dot_general(rq, E, dn, preferred_element_type=jnp.float32)
    pTq = lax.dot_general(T, rq, dn, preferred_element_type=jnp.float32)

    base = nH + nE - 2.0 * GH
    s1 = jnp.zeros_like(base)
    for rel in range(NR):
        nr = jnp.sum(R[rel : rel + 1, :] ** 2)
        d2 = base + (2.0 * PH[:, rel : rel + 1] + nr) - 2.0 * PE[rel : rel + 1, :]
        s1 = s1 - jnp.sqrt(jnp.maximum(d2, 0.0))

    d2q = (nE + 2.0 * pEq) + (nq + nT - 2.0 * pTq) - 2.0 * GT
    s2 = -jnp.sqrt(jnp.maximum(d2q, 0.0))

    score = jnp.sum(s1 * s2, axis=1, keepdims=True) * (1.0 / N)

    dv = H + rq - T
    base_out = -jnp.sqrt(jnp.sum(dv * dv, axis=1, keepdims=True))

    d = depth_ref[0]
    out_ref[...] = jnp.where(d == 0, base_out, score)


def kernel(ent_emb, rule_emb, query_relation, head, tail, depth):
    B = head.shape[0]
    out = pl.pallas_call(
        _score_body,
        out_shape=jax.ShapeDtypeStruct((B, 1), jnp.float32),
        in_specs=[
            pl.BlockSpec(memory_space=pltpu.VMEM),
            pl.BlockSpec(memory_space=pltpu.VMEM),
            pl.BlockSpec(memory_space=pltpu.SMEM),
            pl.BlockSpec(memory_space=pltpu.SMEM),
            pl.BlockSpec(memory_space=pltpu.VMEM),
            pl.BlockSpec(memory_space=pltpu.VMEM),
        ],
        out_specs=pl.BlockSpec(memory_space=pltpu.VMEM),
    )(
        ent_emb,
        rule_emb,
        query_relation.astype(jnp.int32),
        jnp.asarray(depth, jnp.int32).reshape(1),
        head.astype(jnp.int32).reshape(B, 1),
        tail.astype(jnp.int32).reshape(B, 1),
    )
    return out.reshape(B)
